# Initial kernel scaffold; baseline (speedup 1.0000x reference)
import jax
import jax.numpy as jnp
from jax.experimental import pallas as pl


def kernel(inputs):
    raise NotImplementedError("write your pallas kernel here")


# speedup vs baseline: 66.6297x; 66.6297x over previous
import functools

import jax
import jax.numpy as jnp
from jax import lax
from jax.experimental import pallas as pl
from jax.experimental.pallas import tpu as pltpu
from jax.experimental.pallas import tpu_sc as plsc

_R, _C = 16384, 26
_N = _R * _C
_D = 1 << 20
_NC, _NS, _L = 2, 16, 16
_SLICE = _D // _NS
_P1 = 6656
_NCHUNKS = _N // _P1
_CHUNK = _N // _NS
_SUBS = _CHUNK // _P1
_HALF = _CHUNK // _NC
_GB = 128
_NGB = _P1 // _GB
_INF = 2147483647

_mesh = plsc.VectorSubcoreMesh(
    core_axis_name="c", subcore_axis_name="s",
    num_cores=_NC, num_subcores=_NS)


def _fori(n, body, init):
    return lax.fori_loop(jnp.int32(0), jnp.int32(n), body, init)


@functools.partial(
    pl.kernel,
    out_type=jax.ShapeDtypeStruct((_N,), jnp.int32),
    mesh=_mesh,
    scratch_types=[
        pltpu.HBM((_D,), jnp.int32),
        pltpu.HBM((_N,), jnp.int32),
        pltpu.HBM((_NS * 8,), jnp.int32),
        pltpu.VMEM((_SLICE,), jnp.int32),
        pltpu.VMEM((_P1,), jnp.int32),
        pltpu.VMEM((_P1,), jnp.int32),
        pltpu.VMEM((_CHUNK,), jnp.int32),
        pltpu.VMEM((_HALF,), jnp.int32),
        pltpu.VMEM((_NS * 8,), jnp.int32),
        pltpu.VMEM((_L,), jnp.int32),
        pltpu.SemaphoreType.DMA,
    ],
    compiler_params=pltpu.CompilerParams(needs_layout_passes=False),
)
def _lookup_kernel(x_hbm, out_hbm, tbl_hbm, pref_hbm, tot_hbm,
                   tbl, abuf, bbuf, pbuf, fhalf, tot, stg, gsem):
    i32 = jnp.int32
    cid = lax.axis_index("c").astype(i32)
    sid = lax.axis_index("s").astype(i32)
    iota = lax.iota(i32, _L)
    zero = i32(0)
    vl = i32(_L)
    slice_base = sid * i32(_SLICE)

    def init_i(i, carry):
        tbl[pl.ds(i * vl, _L)] = jnp.full((_L,), _INF, i32)
        return carry
    _fori(_SLICE // _L, init_i, zero)

    def chunk_c(c, carry):
        cbase = c * i32(_P1)
        pltpu.sync_copy(x_hbm.at[pl.ds(cbase, _P1)], abuf)

        def vreg_i(i, inner):
            k = abuf[pl.ds(i * vl, _L)]
            lidx = k - slice_base
            m = (lidx >= zero) & (lidx < i32(_SLICE))
            jv = cbase + i * vl + iota
            cur = plsc.load_gather(tbl, [lidx], mask=m)
            win = (m & (jv < cur)).astype(i32)

            def w_cond(w):
                return jnp.max(w) > zero

            def w_body(w):
                plsc.store_scatter(tbl, [lidx], jv, mask=w > zero)
                cur2 = plsc.load_gather(tbl, [lidx], mask=m)
                return (m & (jv < cur2)).astype(i32)

            lax.while_loop(w_cond, w_body, win)
            return inner
        _fori(_P1 // _L, vreg_i, zero)
        return carry
    _fori(_NCHUNKS, chunk_c, zero)

    pltpu.sync_copy(tbl, tbl_hbm.at[pl.ds(slice_base, _SLICE)])
    plsc.subcore_barrier()

    run = zero
    chunk_base = sid * i32(_CHUNK)
    for s in range(_SUBS):
        sbase = chunk_base + i32(s * _P1)
        pltpu.sync_copy(x_hbm.at[pl.ds(sbase, _P1)], abuf)
        cps = [pltpu.async_copy(
                   tbl_hbm.at[abuf.at[pl.ds(b * _GB, _GB)]],
                   bbuf.at[pl.ds(b * _GB, _GB)], gsem)
               for b in range(_NGB)]
        for cp in cps:
            cp.wait()

        def flag_i(i, r, _sbase=sbase, _s=s):
            f = bbuf[pl.ds(i * vl, _L)]
            pos = _sbase + i * vl + iota
            flg = (f == pos).astype(i32)
            pc = plsc.cumsum(flg)
            pbuf[pl.ds(i32(_s * _P1) + i * vl, _L)] = pc + r
            return (r + jnp.sum(flg, dtype=i32)).astype(i32)
        run = _fori(_P1 // _L, flag_i, run)

        my = cid == i32(s // 2)

        @pl.when(my)
        def _(_s=s):
            off = i32((_s % 2) * _P1)

            def cp_i(i, carry):
                fhalf[pl.ds(off + i * vl, _L)] = bbuf[pl.ds(i * vl, _L)]
                return carry
            _fori(_P1 // _L, cp_i, zero)

    stg[...] = jnp.full((_L,), run, i32)
    pltpu.sync_copy(stg.at[pl.ds(0, 8)], tot_hbm.at[pl.ds(sid * i32(8), 8)])
    plsc.subcore_barrier()

    pltpu.sync_copy(tot_hbm, tot)
    t16 = plsc.load_gather(tot, [iota * i32(8)])
    excl = jnp.sum(jnp.where(iota < sid, t16, zero), dtype=i32).astype(i32)

    def add_i(i, carry):
        pbuf[pl.ds(i * vl, _L)] = pbuf[pl.ds(i * vl, _L)] + excl
        return carry
    _fori(_CHUNK // _L, add_i, zero)
    pltpu.sync_copy(pbuf, pref_hbm.at[pl.ds(chunk_base, _CHUNK)])
    plsc.subcore_barrier()

    outbase = chunk_base + cid * i32(_HALF)
    for h in range(_HALF // _P1):
        cps = [pltpu.async_copy(
                   pref_hbm.at[fhalf.at[pl.ds(h * _P1 + b * _GB, _GB)]],
                   bbuf.at[pl.ds(b * _GB, _GB)], gsem)
               for b in range(_NGB)]
        for cp in cps:
            cp.wait()
        pltpu.sync_copy(bbuf,
                        out_hbm.at[pl.ds(outbase + i32(h * _P1), _P1)])


def kernel(inputs):
    x = jnp.reshape(inputs, (-1,)).astype(jnp.int32)
    out = _lookup_kernel(x)
    return jnp.reshape(out, inputs.shape).astype(jnp.int64)


# speedup vs baseline: 128.6113x; 1.9302x over previous
import functools

import jax
import jax.numpy as jnp
from jax import lax
from jax.experimental import pallas as pl
from jax.experimental.pallas import tpu as pltpu
from jax.experimental.pallas import tpu_sc as plsc

_R, _C = 16384, 26
_N = _R * _C
_D = 1 << 20
_NC, _NS, _L = 2, 16, 16
_SLICE = _D // _NS
_P1 = 6656
_NCHUNKS = _N // _P1
_CHUNK = _N // _NS
_SUBS = _CHUNK // _P1
_HALF = _CHUNK // _NC
_GB = 128
_NGB = _P1 // _GB
_INF = 2147483647

_mesh = plsc.VectorSubcoreMesh(
    core_axis_name="c", subcore_axis_name="s",
    num_cores=_NC, num_subcores=_NS)


def _fori(n, body, init):
    return lax.fori_loop(jnp.int32(0), jnp.int32(n), body, init)


@functools.partial(
    pl.kernel,
    out_type=jax.ShapeDtypeStruct((_N,), jnp.int32),
    mesh=_mesh,
    scratch_types=[
        pltpu.HBM((_D,), jnp.int32),
        pltpu.HBM((_N,), jnp.int32),
        pltpu.HBM((_NS * 8,), jnp.int32),
        pltpu.VMEM((_SLICE,), jnp.int32),
        pltpu.VMEM((_P1,), jnp.int32),
        pltpu.VMEM((_P1,), jnp.int32),
        pltpu.VMEM((_CHUNK,), jnp.int32),
        pltpu.VMEM((_HALF,), jnp.int32),
        pltpu.VMEM((_NS * 8,), jnp.int32),
        pltpu.VMEM((_L,), jnp.int32),
        pltpu.SemaphoreType.DMA,
    ],
    compiler_params=pltpu.CompilerParams(needs_layout_passes=False),
)
def _lookup_kernel(x_hbm, out_hbm, tbl_hbm, pref_hbm, tot_hbm,
                   tbl, abuf, bbuf, pbuf, fhalf, tot, stg, gsem):
    i32 = jnp.int32
    cid = lax.axis_index("c").astype(i32)
    sid = lax.axis_index("s").astype(i32)
    iota = lax.iota(i32, _L)
    zero = i32(0)
    vl = i32(_L)
    slice_base = sid * i32(_SLICE)

    def init_i(i, carry):
        tbl[pl.ds(i * vl, _L)] = jnp.full((_L,), _INF, i32)
        return carry
    _fori(_SLICE // _L, init_i, zero)

    riota = i32(_L - 1) - iota
    uslice = jnp.uint32(_SLICE)
    _U = 4

    def chunk_c(c, carry):
        cbase = (i32(_NCHUNKS - 1) - c) * i32(_P1)
        pltpu.sync_copy(x_hbm.at[pl.ds(cbase, _P1)], abuf)

        def vreg_g(g, inner):
            gbase = (i32(_P1 // _L // _U - 1) - g) * i32(_U * _L)
            for u in range(_U - 1, -1, -1):
                vbase = gbase + i32(u * _L)
                k = abuf[pl.ds(vbase, _L)]
                kr = lax.rev(k, (0,))
                lidx = kr - slice_base
                m = plsc.bitcast(lidx, jnp.uint32) < uslice
                jvr = cbase + vbase + riota
                _, lastm = plsc.scan_count(kr, mask=m)
                plsc.store_scatter(tbl, [lidx], jvr, mask=m & lastm)
            return inner
        _fori(_P1 // _L // _U, vreg_g, zero)
        return carry
    _fori(_NCHUNKS, chunk_c, zero)

    pltpu.sync_copy(tbl, tbl_hbm.at[pl.ds(slice_base, _SLICE)])
    plsc.subcore_barrier()

    run = zero
    chunk_base = sid * i32(_CHUNK)
    for s in range(_SUBS):
        sbase = chunk_base + i32(s * _P1)
        pltpu.sync_copy(x_hbm.at[pl.ds(sbase, _P1)], abuf)
        cps = [pltpu.async_copy(
                   tbl_hbm.at[abuf.at[pl.ds(b * _GB, _GB)]],
                   bbuf.at[pl.ds(b * _GB, _GB)], gsem)
               for b in range(_NGB)]
        for cp in cps:
            cp.wait()

        def flag_i(i, r, _sbase=sbase, _s=s):
            f = bbuf[pl.ds(i * vl, _L)]
            pos = _sbase + i * vl + iota
            flg = (f == pos).astype(i32)
            pc = plsc.cumsum(flg)
            pbuf[pl.ds(i32(_s * _P1) + i * vl, _L)] = pc + r
            return (r + jnp.sum(flg, dtype=i32)).astype(i32)
        run = _fori(_P1 // _L, flag_i, run)

        my = cid == i32(s // 2)

        @pl.when(my)
        def _(_s=s):
            off = i32((_s % 2) * _P1)

            def cp_i(i, carry):
                fhalf[pl.ds(off + i * vl, _L)] = bbuf[pl.ds(i * vl, _L)]
                return carry
            _fori(_P1 // _L, cp_i, zero)

    stg[...] = jnp.full((_L,), run, i32)
    pltpu.sync_copy(stg.at[pl.ds(0, 8)], tot_hbm.at[pl.ds(sid * i32(8), 8)])
    plsc.subcore_barrier()

    pltpu.sync_copy(tot_hbm, tot)
    t16 = plsc.load_gather(tot, [iota * i32(8)])
    excl = jnp.sum(jnp.where(iota < sid, t16, zero), dtype=i32).astype(i32)

    def add_i(i, carry):
        pbuf[pl.ds(i * vl, _L)] = pbuf[pl.ds(i * vl, _L)] + excl
        return carry
    _fori(_CHUNK // _L, add_i, zero)
    pltpu.sync_copy(pbuf, pref_hbm.at[pl.ds(chunk_base, _CHUNK)])
    plsc.subcore_barrier()

    outbase = chunk_base + cid * i32(_HALF)
    for h in range(_HALF // _P1):
        cps = [pltpu.async_copy(
                   pref_hbm.at[fhalf.at[pl.ds(h * _P1 + b * _GB, _GB)]],
                   bbuf.at[pl.ds(b * _GB, _GB)], gsem)
               for b in range(_NGB)]
        for cp in cps:
            cp.wait()
        pltpu.sync_copy(bbuf,
                        out_hbm.at[pl.ds(outbase + i32(h * _P1), _P1)])


def kernel(inputs):
    x = jnp.reshape(inputs, (-1,)).astype(jnp.int32)
    out = _lookup_kernel(x)
    return jnp.reshape(out, inputs.shape).astype(jnp.int64)


# speedup vs baseline: 215.5045x; 1.6756x over previous
import functools

import jax
import jax.numpy as jnp
from jax import lax
from jax.experimental import pallas as pl
from jax.experimental.pallas import tpu as pltpu
from jax.experimental.pallas import tpu_sc as plsc

_R, _C = 16384, 26
_N = _R * _C
_D = 1 << 20
_NC, _NS, _L = 2, 16, 16
_SLICE = _D // _NS
_P1 = 6656
_NCHUNKS = _N // _P1
_CHUNK = _N // _NS
_SUBS = _CHUNK // _P1
_HALF = _CHUNK // _NC
_GB = 128
_NGB = _P1 // _GB
_INF = 2147483647

_mesh = plsc.VectorSubcoreMesh(
    core_axis_name="c", subcore_axis_name="s",
    num_cores=_NC, num_subcores=_NS)


def _fori(n, body, init):
    return lax.fori_loop(jnp.int32(0), jnp.int32(n), body, init)


@functools.partial(
    pl.kernel,
    out_type=jax.ShapeDtypeStruct((_N,), jnp.int32),
    mesh=_mesh,
    scratch_types=[
        pltpu.HBM((_D,), jnp.int32),
        pltpu.HBM((_N,), jnp.int32),
        pltpu.HBM((_NS * 8,), jnp.int32),
        pltpu.VMEM((_SLICE,), jnp.int32),
        pltpu.VMEM((_P1,), jnp.int32),
        pltpu.VMEM((_P1,), jnp.int32),
        pltpu.VMEM((_CHUNK,), jnp.int32),
        pltpu.VMEM((_HALF,), jnp.int32),
        pltpu.VMEM((_NS * 8,), jnp.int32),
        pltpu.VMEM((_L,), jnp.int32),
        pltpu.SemaphoreType.DMA,
    ],
    compiler_params=pltpu.CompilerParams(needs_layout_passes=False),
)
def _lookup_kernel(x_hbm, out_hbm, tbl_hbm, pref_hbm, tot_hbm,
                   tbl, abuf, bbuf, pbuf, fhalf, tot, stg, gsem):
    i32 = jnp.int32
    cid = lax.axis_index("c").astype(i32)
    sid = lax.axis_index("s").astype(i32)
    iota = lax.iota(i32, _L)
    zero = i32(0)
    vl = i32(_L)
    slice_base = sid * i32(_SLICE)

    def init_i(i, carry):
        tbl[pl.ds(i * vl, _L)] = jnp.full((_L,), _INF, i32)
        return carry
    _fori(_SLICE // _L, init_i, zero)

    riota = i32(_L - 1) - iota
    uslice = jnp.uint32(_SLICE)
    _U = 4

    def chunk_c(c, carry):
        cbase = (i32(_NCHUNKS - 1) - c) * i32(_P1)
        pltpu.sync_copy(x_hbm.at[pl.ds(cbase, _P1)], abuf)

        def vreg_g(g, inner):
            gbase = (i32(_P1 // _L // _U - 1) - g) * i32(_U * _L)
            pend = []
            for u in range(_U - 1, -1, -1):
                vbase = gbase + i32(u * _L)
                k = abuf[pl.ds(vbase, _L)]
                kr = lax.rev(k, (0,))
                lidx = kr - slice_base
                m = plsc.bitcast(lidx, jnp.uint32) < uslice
                jvr = cbase + vbase + riota
                _, lastm = plsc.scan_count(kr, mask=m)
                pend.append((lidx, jvr, m & lastm))
            for lidx, jvr, msk in pend:
                plsc.store_scatter(tbl, [lidx], jvr, mask=msk)
            return inner
        _fori(_P1 // _L // _U, vreg_g, zero)
        return carry
    _fori(_NCHUNKS, chunk_c, zero)

    pltpu.sync_copy(tbl, tbl_hbm.at[pl.ds(slice_base, _SLICE)])
    plsc.subcore_barrier()

    run = zero
    chunk_base = sid * i32(_CHUNK)
    for s in range(_SUBS):
        sbase = chunk_base + i32(s * _P1)
        pltpu.sync_copy(x_hbm.at[pl.ds(sbase, _P1)], abuf)
        cps = [pltpu.async_copy(
                   tbl_hbm.at[abuf.at[pl.ds(b * _GB, _GB)]],
                   bbuf.at[pl.ds(b * _GB, _GB)], gsem)
               for b in range(_NGB)]
        for cp in cps:
            cp.wait()

        def flag_i(i, r, _sbase=sbase, _s=s):
            f = bbuf[pl.ds(i * vl, _L)]
            pos = _sbase + i * vl + iota
            flg = (f == pos).astype(i32)
            pc = plsc.cumsum(flg)
            pbuf[pl.ds(i32(_s * _P1) + i * vl, _L)] = pc + r
            return (r + jnp.sum(flg, dtype=i32)).astype(i32)
        run = _fori(_P1 // _L, flag_i, run)

        my = cid == i32(s // 2)

        @pl.when(my)
        def _(_s=s):
            off = i32((_s % 2) * _P1)

            def cp_i(i, carry):
                fhalf[pl.ds(off + i * vl, _L)] = bbuf[pl.ds(i * vl, _L)]
                return carry
            _fori(_P1 // _L, cp_i, zero)

    stg[...] = jnp.full((_L,), run, i32)
    pltpu.sync_copy(stg.at[pl.ds(0, 8)], tot_hbm.at[pl.ds(sid * i32(8), 8)])
    plsc.subcore_barrier()

    pltpu.sync_copy(tot_hbm, tot)
    t16 = plsc.load_gather(tot, [iota * i32(8)])
    excl = jnp.sum(jnp.where(iota < sid, t16, zero), dtype=i32).astype(i32)

    def add_i(i, carry):
        pbuf[pl.ds(i * vl, _L)] = pbuf[pl.ds(i * vl, _L)] + excl
        return carry
    _fori(_CHUNK // _L, add_i, zero)
    pltpu.sync_copy(pbuf, pref_hbm.at[pl.ds(chunk_base, _CHUNK)])
    plsc.subcore_barrier()

    outbase = chunk_base + cid * i32(_HALF)
    for h in range(_HALF // _P1):
        cps = [pltpu.async_copy(
                   pref_hbm.at[fhalf.at[pl.ds(h * _P1 + b * _GB, _GB)]],
                   bbuf.at[pl.ds(b * _GB, _GB)], gsem)
               for b in range(_NGB)]
        for cp in cps:
            cp.wait()
        pltpu.sync_copy(bbuf,
                        out_hbm.at[pl.ds(outbase + i32(h * _P1), _P1)])


def kernel(inputs):
    x = jnp.reshape(inputs, (-1,)).astype(jnp.int32)
    out = _lookup_kernel(x)
    return jnp.reshape(out, inputs.shape).astype(jnp.int64)


# speedup vs baseline: 238.9024x; 1.1086x over previous
import functools

import jax
import jax.numpy as jnp
from jax import lax
from jax.experimental import pallas as pl
from jax.experimental.pallas import tpu as pltpu
from jax.experimental.pallas import tpu_sc as plsc

_R, _C = 16384, 26
_N = _R * _C
_D = 1 << 20
_NC, _NS, _L = 2, 16, 16
_SLICE = _D // _NS
_P1 = 6656
_NCHUNKS = _N // _P1
_CHUNK = _N // _NS
_SUBS = _CHUNK // _P1
_HALF = _CHUNK // _NC
_GB = 128
_NGB = _P1 // _GB
_INF = 2147483647

_mesh = plsc.VectorSubcoreMesh(
    core_axis_name="c", subcore_axis_name="s",
    num_cores=_NC, num_subcores=_NS)


def _fori(n, body, init):
    return lax.fori_loop(jnp.int32(0), jnp.int32(n), body, init)


@functools.partial(
    pl.kernel,
    out_type=jax.ShapeDtypeStruct((_N,), jnp.int32),
    mesh=_mesh,
    scratch_types=[
        pltpu.HBM((_D,), jnp.int32),
        pltpu.HBM((_N,), jnp.int32),
        pltpu.HBM((_NS * 8,), jnp.int32),
        pltpu.VMEM((_SLICE,), jnp.int32),
        pltpu.VMEM((_P1,), jnp.int32),
        pltpu.VMEM((_P1,), jnp.int32),
        pltpu.VMEM((_CHUNK,), jnp.int32),
        pltpu.VMEM((_HALF,), jnp.int32),
        pltpu.VMEM((_NS * 8,), jnp.int32),
        pltpu.VMEM((_L,), jnp.int32),
        pltpu.VMEM((_CHUNK // _L,), jnp.int32),
        pltpu.SemaphoreType.DMA,
        pltpu.SemaphoreType.DMA,
    ],
    compiler_params=pltpu.CompilerParams(needs_layout_passes=False),
)
def _lookup_kernel(x_hbm, out_hbm, tbl_hbm, pref_hbm, tot_hbm,
                   tbl, abuf, bbuf, pbuf, fhalf, tot, stg, vtot, gsem, ssem):
    i32 = jnp.int32
    cid = lax.axis_index("c").astype(i32)
    sid = lax.axis_index("s").astype(i32)
    iota = lax.iota(i32, _L)
    zero = i32(0)
    vl = i32(_L)
    slice_base = sid * i32(_SLICE)

    def init_i(i, carry):
        tbl[pl.ds(i * vl, _L)] = jnp.full((_L,), _INF, i32)
        return carry
    _fori(_SLICE // _L, init_i, zero)

    riota = i32(_L - 1) - iota
    uslice = jnp.uint32(_SLICE)
    _U = 4

    def scan_buf(buf, cbase):
        def vreg_g(g, inner):
            gbase = (i32(_P1 // _L // _U - 1) - g) * i32(_U * _L)
            pend = []
            for u in range(_U - 1, -1, -1):
                vbase = gbase + i32(u * _L)
                k = buf[pl.ds(vbase, _L)]
                kr = lax.rev(k, (0,))
                lidx = kr - slice_base
                m = plsc.bitcast(lidx, jnp.uint32) < uslice
                jvr = cbase + vbase + riota
                _, lastm = plsc.scan_count(kr, mask=m)
                pend.append((lidx, jvr, m & lastm))
            for lidx, jvr, msk in pend:
                plsc.store_scatter(tbl, [lidx], jvr, mask=msk)
            return inner
        _fori(_P1 // _L // _U, vreg_g, zero)

    pltpu.async_copy(
        x_hbm.at[pl.ds(i32((_NCHUNKS - 1) * _P1), _P1)], abuf, ssem)

    def pair_p(p, carry):
        base_a = (i32(_NCHUNKS - 1) - p * i32(2)) * i32(_P1)
        base_b = base_a - i32(_P1)
        pltpu.make_async_copy(x_hbm.at[pl.ds(base_a, _P1)], abuf, ssem).wait()
        pltpu.async_copy(x_hbm.at[pl.ds(base_b, _P1)], bbuf, ssem)
        scan_buf(abuf, base_a)
        pltpu.make_async_copy(x_hbm.at[pl.ds(base_b, _P1)], bbuf, ssem).wait()

        @pl.when(p < i32(_NCHUNKS // 2 - 1))
        def _():
            pltpu.async_copy(
                x_hbm.at[pl.ds(base_b - i32(_P1), _P1)], abuf, ssem)
        scan_buf(bbuf, base_b)
        return carry
    _fori(_NCHUNKS // 2, pair_p, zero)

    pltpu.sync_copy(tbl, tbl_hbm.at[pl.ds(slice_base, _SLICE)])
    plsc.subcore_barrier()

    chunk_base = sid * i32(_CHUNK)
    lane0 = iota == zero
    for s in range(_SUBS):
        sbase = chunk_base + i32(s * _P1)
        pltpu.sync_copy(x_hbm.at[pl.ds(sbase, _P1)], abuf)
        cps = [pltpu.async_copy(
                   tbl_hbm.at[abuf.at[pl.ds(b * _GB, _GB)]],
                   bbuf.at[pl.ds(b * _GB, _GB)], gsem)
               for b in range(_NGB)]
        for cp in cps:
            cp.wait()

        def flag_i(i, carry, _sbase=sbase, _s=s):
            f = bbuf[pl.ds(i * vl, _L)]
            pos = _sbase + i * vl + iota
            flg = (f == pos).astype(i32)
            pc = plsc.cumsum(flg)
            pbuf[pl.ds(i32(_s * _P1) + i * vl, _L)] = pc
            tv = jnp.full((_L,), jnp.sum(flg, dtype=i32), i32)
            tidx = jnp.full((_L,), i + i32(_s * (_P1 // _L)), i32)
            plsc.store_scatter(vtot, [tidx], tv, mask=lane0)
            return carry
        _fori(_P1 // _L, flag_i, zero)

        my = cid == i32(s // 2)

        @pl.when(my)
        def _(_s=s):
            off = i32((_s % 2) * _P1)

            def cp_i(i, carry):
                fhalf[pl.ds(off + i * vl, _L)] = bbuf[pl.ds(i * vl, _L)]
                return carry
            _fori(_P1 // _L, cp_i, zero)

    def off_i(i, r):
        v = vtot[pl.ds(i * vl, _L)]
        pc2 = plsc.cumsum(v)
        vtot[pl.ds(i * vl, _L)] = pc2 - v + r
        return (r + jnp.sum(v, dtype=i32)).astype(i32)
    run = _fori(_CHUNK // _L // _L, off_i, zero)

    stg[...] = jnp.full((_L,), run, i32)
    pltpu.sync_copy(stg.at[pl.ds(0, 8)], tot_hbm.at[pl.ds(sid * i32(8), 8)])
    plsc.subcore_barrier()

    pltpu.sync_copy(tot_hbm, tot)
    t16 = plsc.load_gather(tot, [iota * i32(8)])
    excl = jnp.sum(jnp.where(iota < sid, t16, zero), dtype=i32).astype(i32)

    def add_i(i, carry):
        off = plsc.load_gather(vtot, [jnp.full((_L,), i, i32)])
        pbuf[pl.ds(i * vl, _L)] = pbuf[pl.ds(i * vl, _L)] + off + excl
        return carry
    _fori(_CHUNK // _L, add_i, zero)
    pltpu.sync_copy(pbuf, pref_hbm.at[pl.ds(chunk_base, _CHUNK)])
    plsc.subcore_barrier()

    outbase = chunk_base + cid * i32(_HALF)
    for h in range(_HALF // _P1):
        cps = [pltpu.async_copy(
                   pref_hbm.at[fhalf.at[pl.ds(h * _P1 + b * _GB, _GB)]],
                   bbuf.at[pl.ds(b * _GB, _GB)], gsem)
               for b in range(_NGB)]
        for cp in cps:
            cp.wait()
        pltpu.sync_copy(bbuf,
                        out_hbm.at[pl.ds(outbase + i32(h * _P1), _P1)])


def kernel(inputs):
    x = jnp.reshape(inputs, (-1,)).astype(jnp.int32)
    out = _lookup_kernel(x)
    return jnp.reshape(out, inputs.shape).astype(jnp.int64)


# speedup vs baseline: 271.6435x; 1.1370x over previous
import functools

import jax
import jax.numpy as jnp
from jax import lax
from jax.experimental import pallas as pl
from jax.experimental.pallas import tpu as pltpu
from jax.experimental.pallas import tpu_sc as plsc

_R, _C = 16384, 26
_N = _R * _C
_D = 1 << 20
_NC, _NS, _L = 2, 16, 16
_SLICE = _D // _NS
_P1 = 6656
_NCHUNKS = _N // _P1
_CHUNK = _N // _NS
_SUBS = _CHUNK // _P1
_HALF = _CHUNK // _NC
_GB = 128
_NGB = _P1 // _GB
_INF = 2147483647

_mesh = plsc.VectorSubcoreMesh(
    core_axis_name="c", subcore_axis_name="s",
    num_cores=_NC, num_subcores=_NS)


def _fori(n, body, init):
    return lax.fori_loop(jnp.int32(0), jnp.int32(n), body, init)


@functools.partial(
    pl.kernel,
    out_type=jax.ShapeDtypeStruct((_N,), jnp.int32),
    mesh=_mesh,
    scratch_types=[
        pltpu.HBM((_D,), jnp.int32),
        pltpu.HBM((_N,), jnp.int32),
        pltpu.HBM((_NS * 8,), jnp.int32),
        pltpu.VMEM((_SLICE,), jnp.int32),
        pltpu.VMEM((_P1,), jnp.int32),
        pltpu.VMEM((_P1,), jnp.int32),
        pltpu.VMEM((_CHUNK,), jnp.int32),
        pltpu.VMEM((_HALF,), jnp.int32),
        pltpu.VMEM((_NS * 8,), jnp.int32),
        pltpu.VMEM((_L,), jnp.int32),
        pltpu.VMEM((_CHUNK // _L,), jnp.int32),
        pltpu.SemaphoreType.DMA,
        pltpu.SemaphoreType.DMA,
    ],
    compiler_params=pltpu.CompilerParams(needs_layout_passes=False),
)
def _lookup_kernel(x_hbm, out_hbm, tbl_hbm, pref_hbm, tot_hbm,
                   tbl, abuf, bbuf, pbuf, fhalf, tot, stg, vtot, gsem, ssem):
    i32 = jnp.int32
    cid = lax.axis_index("c").astype(i32)
    sid = lax.axis_index("s").astype(i32)
    iota = lax.iota(i32, _L)
    zero = i32(0)
    vl = i32(_L)
    slice_base = sid * i32(_SLICE)

    def init_i(i, carry):
        tbl[pl.ds(i * vl, _L)] = jnp.full((_L,), _INF, i32)
        return carry
    _fori(_SLICE // _L, init_i, zero)

    riota = i32(_L - 1) - iota
    uslice = jnp.uint32(_SLICE)
    _U = 8

    def scan_buf(buf, cbase):
        def vreg_g(g, inner):
            gbase = (i32(_P1 // _L // _U - 1) - g) * i32(_U * _L)
            pend = []
            for u in range(_U - 1, -1, -1):
                vbase = gbase + i32(u * _L)
                k = buf[pl.ds(vbase, _L)]
                kr = lax.rev(k, (0,))
                lidx = kr - slice_base
                m = plsc.bitcast(lidx, jnp.uint32) < uslice
                jvr = cbase + vbase + riota
                _, lastm = plsc.scan_count(kr, mask=m)
                pend.append((lidx, jvr, m & lastm))
            for lidx, jvr, msk in pend:
                plsc.store_scatter(tbl, [lidx], jvr, mask=msk)
            return inner
        _fori(_P1 // _L // _U, vreg_g, zero)

    pltpu.async_copy(
        x_hbm.at[pl.ds(i32((_NCHUNKS - 1) * _P1), _P1)], abuf, ssem)

    def pair_p(p, carry):
        base_a = (i32(_NCHUNKS - 1) - p * i32(2)) * i32(_P1)
        base_b = base_a - i32(_P1)
        pltpu.make_async_copy(x_hbm.at[pl.ds(base_a, _P1)], abuf, ssem).wait()
        pltpu.async_copy(x_hbm.at[pl.ds(base_b, _P1)], bbuf, ssem)
        scan_buf(abuf, base_a)
        pltpu.make_async_copy(x_hbm.at[pl.ds(base_b, _P1)], bbuf, ssem).wait()

        @pl.when(p < i32(_NCHUNKS // 2 - 1))
        def _():
            pltpu.async_copy(
                x_hbm.at[pl.ds(base_b - i32(_P1), _P1)], abuf, ssem)
        scan_buf(bbuf, base_b)
        return carry
    _fori(_NCHUNKS // 2, pair_p, zero)

    pltpu.sync_copy(tbl, tbl_hbm.at[pl.ds(slice_base, _SLICE)])
    plsc.subcore_barrier()

    chunk_base = sid * i32(_CHUNK)
    lane0 = iota == zero
    for s in range(_SUBS):
        sbase = chunk_base + i32(s * _P1)
        pltpu.sync_copy(x_hbm.at[pl.ds(sbase, _P1)], abuf)
        cps = [pltpu.async_copy(
                   tbl_hbm.at[abuf.at[pl.ds(b * _GB, _GB)]],
                   bbuf.at[pl.ds(b * _GB, _GB)], gsem)
               for b in range(_NGB)]
        for cp in cps:
            cp.wait()

        def flag_i(i, carry, _sbase=sbase, _s=s):
            f = bbuf[pl.ds(i * vl, _L)]
            pos = _sbase + i * vl + iota
            flg = (f == pos).astype(i32)
            pc = plsc.cumsum(flg)
            pbuf[pl.ds(i32(_s * _P1) + i * vl, _L)] = pc
            tv = jnp.full((_L,), jnp.sum(flg, dtype=i32), i32)
            tidx = jnp.full((_L,), i + i32(_s * (_P1 // _L)), i32)
            plsc.store_scatter(vtot, [tidx], tv, mask=lane0)
            return carry
        _fori(_P1 // _L, flag_i, zero)

        my = cid == i32(s // 2)

        @pl.when(my)
        def _(_s=s):
            off = i32((_s % 2) * _P1)

            def cp_i(i, carry):
                fhalf[pl.ds(off + i * vl, _L)] = bbuf[pl.ds(i * vl, _L)]
                return carry
            _fori(_P1 // _L, cp_i, zero)

    def off_i(i, r):
        v = vtot[pl.ds(i * vl, _L)]
        pc2 = plsc.cumsum(v)
        vtot[pl.ds(i * vl, _L)] = pc2 - v + r
        return (r + jnp.sum(v, dtype=i32)).astype(i32)
    run = _fori(_CHUNK // _L // _L, off_i, zero)

    stg[...] = jnp.full((_L,), run, i32)
    pltpu.sync_copy(stg.at[pl.ds(0, 8)], tot_hbm.at[pl.ds(sid * i32(8), 8)])
    plsc.subcore_barrier()

    pltpu.sync_copy(tot_hbm, tot)
    t16 = plsc.load_gather(tot, [iota * i32(8)])
    excl = jnp.sum(jnp.where(iota < sid, t16, zero), dtype=i32).astype(i32)

    def add_i(i, carry):
        off = plsc.load_gather(vtot, [jnp.full((_L,), i, i32)])
        pbuf[pl.ds(i * vl, _L)] = pbuf[pl.ds(i * vl, _L)] + off + excl
        return carry
    _fori(_CHUNK // _L, add_i, zero)
    pltpu.sync_copy(pbuf, pref_hbm.at[pl.ds(chunk_base, _CHUNK)])
    plsc.subcore_barrier()

    outbase = chunk_base + cid * i32(_HALF)
    for h in range(_HALF // _P1):
        cps = [pltpu.async_copy(
                   pref_hbm.at[fhalf.at[pl.ds(h * _P1 + b * _GB, _GB)]],
                   bbuf.at[pl.ds(b * _GB, _GB)], gsem)
               for b in range(_NGB)]
        for cp in cps:
            cp.wait()
        pltpu.sync_copy(bbuf,
                        out_hbm.at[pl.ds(outbase + i32(h * _P1), _P1)])


def kernel(inputs):
    x = jnp.reshape(inputs, (-1,)).astype(jnp.int32)
    out = _lookup_kernel(x)
    return jnp.reshape(out, inputs.shape).astype(jnp.int64)
